# baseline (speedup 1.0000x reference)
import jax
import jax.numpy as jnp
from jax import lax
from jax.experimental import pallas as pl
from jax.experimental.pallas import tpu as pltpu
from jax.experimental.pallas import tpu_sc as plsc

HIDDEN = 64
SCALE = 8.0

_NC = 2
_NS = 16
_NW = _NC * _NS

_C = 128
_B = 4096 * 200
_BPW = _B // _NW
_NCH = _BPW // _C
_D = 4


def _body(x_hbm, table_hbm, out_hbm, *scratch):
    idxs = scratch[0:_D]
    rows = scratch[_D:2 * _D]
    semg = scratch[2 * _D:3 * _D]
    semo = scratch[3 * _D:4 * _D]

    wid = lax.axis_index("s") * _NC + lax.axis_index("c")
    base = wid * _BPW

    def out_slice(c):
        return out_hbm.at[pl.ds(base + c * _C, _C)]

    def start(c, b):
        pltpu.sync_copy(x_hbm.at[wid, c], idxs[b])
        pltpu.async_copy(table_hbm.at[idxs[b]], rows[b], semg[b])

    def quad_body(cc, carry):
        c0 = cc * 4
        for j in range(4):
            c = c0 + j
            b = j
            bn = (j + 3) % 4

            @pl.when(c + 3 < _NCH)
            def _prefetch(c=c, bn=bn):
                @pl.when(c >= 1)
                def _(c=c, bn=bn):
                    pltpu.make_async_copy(
                        rows[bn], out_slice(c - 1), semo[bn]
                    ).wait()

                start(c + 3, bn)

            pltpu.make_async_copy(
                table_hbm.at[idxs[b]], rows[b], semg[b]
            ).wait()
            pltpu.async_copy(rows[b], out_slice(c), semo[b])
        return carry

    for c in range(3):
        start(c, c)

    lax.fori_loop(0, _NCH // 4, quad_body, 0)

    for j in range(4):
        c = _NCH - 4 + j
        pltpu.make_async_copy(rows[c % 4], out_slice(c), semo[c % 4]).wait()


def kernel(x, shared_weights):
    b_total = x.size
    assert b_total == _B

    xr = x.astype(jnp.int32).reshape(_NW, _NCH, _C)

    wpad = jnp.pad(shared_weights, ((0, 0), (0, 128 - HIDDEN)))
    row_ids = lax.broadcasted_iota(jnp.int32, wpad.shape, 0)
    wprep = jnp.where(row_ids == 0, jnp.float32(0.0),
                      wpad * jnp.float32(SCALE))

    mesh = plsc.VectorSubcoreMesh(core_axis_name="c", subcore_axis_name="s")
    run = pl.kernel(
        _body,
        out_type=jax.ShapeDtypeStruct((_B, 128), jnp.float32),
        mesh=mesh,
        scratch_types=(
            [pltpu.VMEM((_C,), jnp.int32) for _ in range(_D)]
            + [pltpu.VMEM((_C, 128), jnp.float32) for _ in range(_D)]
            + [pltpu.SemaphoreType.DMA for _ in range(_D)]
            + [pltpu.SemaphoreType.DMA for _ in range(_D)]
        ),
        compiler_params=pltpu.CompilerParams(use_tc_tiling_on_sc=False),
    )
    out = run(xr, wprep)
    return out.reshape(4096, 200, 128)[:, :, :HIDDEN]
